# baseline (speedup 1.0000x reference)
import jax
import jax.numpy as jnp
from jax.experimental import pallas as pl
from jax.experimental.pallas import tpu as pltpu

G, S, B, D, FF, OUT = 2, 2048, 128, 1024, 4096, 512
N = S * B
MBLK = 2048
D2 = 2 * D


def _extract_kernel(wcat_ref, gidf_ref, xf_ref, yf_ref, tf_ref,
                    out_ref, inpt_ref, a0_ref, a1_ref, hacc_ref):
    mi = pl.program_id(0)
    nm = pl.num_programs(0) - 1

    @pl.when(mi == 0)
    def _prologue():
        m0 = jnp.where(gidf_ref[...] == 0, 1.0, 0.0)
        xr, yr, tr = xf_ref[...], yf_ref[...], tf_ref[...]
        x0 = xr * m0
        y0 = yr * m0
        t0 = tr * m0
        inpt_ref[...] = jnp.concatenate(
            [x0, xr - x0, y0, yr - y0, t0, tr - t0, m0, 1.0 - m0],
            axis=0).astype(jnp.bfloat16)
        a1_ref[...] = jnp.zeros((D2, MBLK), jnp.float32)
        hacc_ref[...] = jnp.zeros((D, B), jnp.float32)

    mim = jnp.minimum(mi, nm - 1)
    lhs = inpt_ref[:, pl.ds(mim * MBLK, MBLK)]

    def _produce(dst_ref):
        dst_ref[...] = jax.lax.dot_general(
            wcat_ref[...], lhs, (((0,), (0,)), ((), ())),
            preferred_element_type=jnp.float32)

    def _consume(src_ref):
        ap = src_ref[...]
        f = jnp.tanh(ap[:D, :]) + jnp.maximum(ap[D:, :], 0.0)
        part = f[:, 0:B]
        for j in range(1, MBLK // B):
            part = part + f[:, j * B:(j + 1) * B]
        hacc_ref[...] += part

    @pl.when(mi % 2 == 0)
    def _even():
        _produce(a0_ref)
        _consume(a1_ref)

    @pl.when(mi % 2 == 1)
    def _odd():
        _produce(a1_ref)
        _consume(a0_ref)

    @pl.when(mi == nm)
    def _emit():
        out_ref[...] = hacc_ref[...] * (1.0 / S)


def _mlp_kernel(h_ref, wm1_ref, bm1_ref, wm2_ref, bm2_ref, wout_ref, out_ref):
    h1 = jax.lax.dot_general(h_ref[...], wm1_ref[...],
                             (((0,), (0,)), ((), ())),
                             preferred_element_type=jnp.float32)
    h1 = jnp.maximum(h1 + bm1_ref[...], 0.0)
    h2 = jnp.dot(h1, wm2_ref[...],
                 preferred_element_type=jnp.float32) + bm2_ref[...]
    out_ref[...] = jnp.dot(h2, wout_ref[...],
                           preferred_element_type=jnp.float32)


@jax.jit
def kernel(x, y, t, grid_ids, W1, b1, W2, b2, Wm1, bm1, Wm2, bm2, Wout):
    top = jnp.stack([W1[0, 0], W1[1, 0], W1[0, 1], W1[1, 1],
                     W1[0, 2], W1[1, 2], b1[0], b1[1]])
    bot = jnp.stack([W2[0, 0], W2[1, 0], W2[0, 1], W2[1, 1],
                     W2[0, 2], W2[1, 2], b2[0], b2[1]])
    wcat = jnp.concatenate([top, bot], axis=1).astype(jnp.bfloat16)

    xf = x.reshape(1, N)
    yf = y.reshape(1, N)
    tf = t.reshape(1, N)
    gidf = jnp.tile(grid_ids, S).reshape(1, N)

    nm = N // MBLK
    h = pl.pallas_call(
        _extract_kernel,
        grid=(nm + 1,),
        in_specs=[
            pl.BlockSpec((8, D2), lambda mi: (0, 0)),
            pl.BlockSpec((1, N), lambda mi: (0, 0)),
            pl.BlockSpec((1, N), lambda mi: (0, 0)),
            pl.BlockSpec((1, N), lambda mi: (0, 0)),
            pl.BlockSpec((1, N), lambda mi: (0, 0)),
        ],
        out_specs=pl.BlockSpec((D, B), lambda mi: (0, 0)),
        out_shape=jax.ShapeDtypeStruct((D, B), jnp.float32),
        scratch_shapes=[
            pltpu.VMEM((8, N), jnp.bfloat16),
            pltpu.VMEM((D2, MBLK), jnp.float32),
            pltpu.VMEM((D2, MBLK), jnp.float32),
            pltpu.VMEM((D, B), jnp.float32),
        ],
    )(wcat, gidf, xf, yf, tf)

    out = pl.pallas_call(
        _mlp_kernel,
        in_specs=[
            pl.BlockSpec((D, B), lambda: (0, 0)),
            pl.BlockSpec((D, FF), lambda: (0, 0)),
            pl.BlockSpec((1, FF), lambda: (0, 0)),
            pl.BlockSpec((FF, D), lambda: (0, 0)),
            pl.BlockSpec((1, D), lambda: (0, 0)),
            pl.BlockSpec((D, OUT), lambda: (0, 0)),
        ],
        out_specs=pl.BlockSpec((B, OUT), lambda: (0, 0)),
        out_shape=jax.ShapeDtypeStruct((B, OUT), jnp.float32),
    )(h, Wm1, bm1.reshape(1, FF), Wm2, bm2.reshape(1, D), Wout)
    return out


# speedup vs baseline: 2.1319x; 2.1319x over previous
import jax
import jax.numpy as jnp
from jax.experimental import pallas as pl
from jax.experimental.pallas import tpu as pltpu

G, S, B, D, FF, OUT = 2, 2048, 128, 1024, 4096, 512
N = S * B
MBLK = 1024


def _fused_kernel(wcat_ref, gidf_ref, xf_ref, yf_ref, tf_ref,
                  wm1_hbm, bm1_ref, wm2_hbm, bm2_ref, wout_hbm,
                  out_ref,
                  inpt_ref, hacc_ref, wm1_v, wm2_v, wout_v,
                  sem1, sem2, sem3):
    mi = pl.program_id(0)
    nm = pl.num_programs(0)

    @pl.when(mi == 0)
    def _prologue():
        pltpu.make_async_copy(wm1_hbm, wm1_v, sem1).start()
        pltpu.make_async_copy(wm2_hbm, wm2_v, sem2).start()
        pltpu.make_async_copy(wout_hbm, wout_v, sem3).start()
        m0 = jnp.where(gidf_ref[...] == 0, 1.0, 0.0)
        xr, yr, tr = xf_ref[...], yf_ref[...], tf_ref[...]
        x0 = xr * m0
        y0 = yr * m0
        t0 = tr * m0
        inpt_ref[...] = jnp.concatenate(
            [x0, xr - x0, y0, yr - y0, t0, tr - t0, m0, 1.0 - m0],
            axis=0).astype(jnp.bfloat16)

    lhs = inpt_ref[:, pl.ds(mi * MBLK, MBLK)]
    a = jax.lax.dot_general(wcat_ref[...], lhs, (((0,), (0,)), ((), ())),
                            preferred_element_type=jnp.float32)
    f = jnp.tanh(a[:D, :]) + jnp.maximum(a[D:, :], 0.0)
    part = f[:, 0:B]
    for j in range(1, MBLK // B):
        part = part + f[:, j * B:(j + 1) * B]

    @pl.when(mi == 0)
    def _init():
        hacc_ref[...] = part

    @pl.when(mi != 0)
    def _acc():
        hacc_ref[...] += part

    @pl.when(mi == nm - 1)
    def _mlp():
        pltpu.make_async_copy(wm1_hbm, wm1_v, sem1).wait()
        pltpu.make_async_copy(wm2_hbm, wm2_v, sem2).wait()
        pltpu.make_async_copy(wout_hbm, wout_v, sem3).wait()
        h = hacc_ref[...] * (1.0 / S)
        h1 = jax.lax.dot_general(h, wm1_v[...], (((0,), (0,)), ((), ())),
                                 preferred_element_type=jnp.float32)
        h1 = jnp.maximum(h1 + bm1_ref[...], 0.0)
        h2 = jnp.dot(h1, wm2_v[...],
                     preferred_element_type=jnp.float32) + bm2_ref[...]
        out_ref[...] = jnp.dot(h2, wout_v[...],
                               preferred_element_type=jnp.float32)


@jax.jit
def kernel(x, y, t, grid_ids, W1, b1, W2, b2, Wm1, bm1, Wm2, bm2, Wout):
    top = jnp.stack([W1[0, 0], W1[1, 0], W1[0, 1], W1[1, 1],
                     W1[0, 2], W1[1, 2], b1[0], b1[1]])
    bot = jnp.stack([W2[0, 0], W2[1, 0], W2[0, 1], W2[1, 1],
                     W2[0, 2], W2[1, 2], b2[0], b2[1]])
    wcat = jnp.concatenate([top, bot], axis=1).astype(jnp.bfloat16)

    xf = x.reshape(1, N)
    yf = y.reshape(1, N)
    tf = t.reshape(1, N)
    gidf = jnp.tile(grid_ids, S).reshape(1, N)

    nm = N // MBLK
    out = pl.pallas_call(
        _fused_kernel,
        grid=(nm,),
        in_specs=[
            pl.BlockSpec((8, 2 * D), lambda mi: (0, 0)),
            pl.BlockSpec((1, N), lambda mi: (0, 0)),
            pl.BlockSpec((1, N), lambda mi: (0, 0)),
            pl.BlockSpec((1, N), lambda mi: (0, 0)),
            pl.BlockSpec((1, N), lambda mi: (0, 0)),
            pl.BlockSpec(memory_space=pl.ANY),
            pl.BlockSpec((1, FF), lambda mi: (0, 0)),
            pl.BlockSpec(memory_space=pl.ANY),
            pl.BlockSpec((1, D), lambda mi: (0, 0)),
            pl.BlockSpec(memory_space=pl.ANY),
        ],
        out_specs=pl.BlockSpec((B, OUT), lambda mi: (0, 0)),
        out_shape=jax.ShapeDtypeStruct((B, OUT), jnp.float32),
        scratch_shapes=[
            pltpu.VMEM((8, N), jnp.bfloat16),
            pltpu.VMEM((D, B), jnp.float32),
            pltpu.VMEM((D, FF), jnp.float32),
            pltpu.VMEM((FF, D), jnp.float32),
            pltpu.VMEM((D, OUT), jnp.float32),
            pltpu.SemaphoreType.DMA,
            pltpu.SemaphoreType.DMA,
            pltpu.SemaphoreType.DMA,
        ],
    )(wcat, gidf, xf, yf, tf, Wm1, bm1.reshape(1, FF), Wm2,
      bm2.reshape(1, D), Wout)
    return out


# speedup vs baseline: 2.2941x; 1.0761x over previous
import jax
import jax.numpy as jnp
from jax.experimental import pallas as pl
from jax.experimental.pallas import tpu as pltpu

G, S, B, D, FF, OUT = 2, 2048, 128, 1024, 4096, 512
N = S * B
MBLK = 2048


def _extract_kernel(wcat_ref, gidf_ref, xf_ref, yf_ref, tf_ref,
                    out_ref, inpt_ref, hacc_ref):
    mi = pl.program_id(0)
    nm = pl.num_programs(0)

    @pl.when(mi == 0)
    def _prologue():
        m0 = jnp.where(gidf_ref[...] == 0, 1.0, 0.0)
        xr, yr, tr = xf_ref[...], yf_ref[...], tf_ref[...]
        x0 = xr * m0
        y0 = yr * m0
        t0 = tr * m0
        inpt_ref[...] = jnp.concatenate(
            [x0, xr - x0, y0, yr - y0, t0, tr - t0, m0, 1.0 - m0],
            axis=0).astype(jnp.bfloat16)

    lhs = inpt_ref[:, pl.ds(mi * MBLK, MBLK)]
    a = jax.lax.dot_general(wcat_ref[...], lhs, (((0,), (0,)), ((), ())),
                            preferred_element_type=jnp.float32)
    f = jnp.tanh(a[:D, :]) + jnp.maximum(a[D:, :], 0.0)
    part = f[:, 0:B]
    for j in range(1, MBLK // B):
        part = part + f[:, j * B:(j + 1) * B]

    @pl.when(mi == 0)
    def _init():
        hacc_ref[...] = part

    @pl.when((mi != 0) & (mi != nm - 1))
    def _acc():
        hacc_ref[...] += part

    @pl.when(mi == nm - 1)
    def _emit():
        out_ref[...] = (hacc_ref[...] + part) * (1.0 / S)


def _mlp_kernel(h_ref, wm1_ref, bm1_ref, wm2_ref, bm2_ref, wout_ref, out_ref):
    h1 = jax.lax.dot_general(h_ref[...], wm1_ref[...],
                             (((0,), (0,)), ((), ())),
                             preferred_element_type=jnp.float32)
    h1 = jnp.maximum(h1 + bm1_ref[...], 0.0)
    h2 = jnp.dot(h1, wm2_ref[...],
                 preferred_element_type=jnp.float32) + bm2_ref[...]
    out_ref[...] = jnp.dot(h2, wout_ref[...],
                           preferred_element_type=jnp.float32)


@jax.jit
def kernel(x, y, t, grid_ids, W1, b1, W2, b2, Wm1, bm1, Wm2, bm2, Wout):
    top = jnp.stack([W1[0, 0], W1[1, 0], W1[0, 1], W1[1, 1],
                     W1[0, 2], W1[1, 2], b1[0], b1[1]])
    bot = jnp.stack([W2[0, 0], W2[1, 0], W2[0, 1], W2[1, 1],
                     W2[0, 2], W2[1, 2], b2[0], b2[1]])
    wcat = jnp.concatenate([top, bot], axis=1).astype(jnp.bfloat16)

    xf = x.reshape(1, N)
    yf = y.reshape(1, N)
    tf = t.reshape(1, N)
    gidf = jnp.tile(grid_ids, S).reshape(1, N)

    nm = N // MBLK
    h = pl.pallas_call(
        _extract_kernel,
        grid=(nm,),
        in_specs=[
            pl.BlockSpec((8, 2 * D), lambda mi: (0, 0)),
            pl.BlockSpec((1, N), lambda mi: (0, 0)),
            pl.BlockSpec((1, N), lambda mi: (0, 0)),
            pl.BlockSpec((1, N), lambda mi: (0, 0)),
            pl.BlockSpec((1, N), lambda mi: (0, 0)),
        ],
        out_specs=pl.BlockSpec((D, B), lambda mi: (0, 0)),
        out_shape=jax.ShapeDtypeStruct((D, B), jnp.float32),
        scratch_shapes=[
            pltpu.VMEM((8, N), jnp.bfloat16),
            pltpu.VMEM((D, B), jnp.float32),
        ],
    )(wcat, gidf, xf, yf, tf)

    out = pl.pallas_call(
        _mlp_kernel,
        in_specs=[
            pl.BlockSpec((D, B), lambda: (0, 0)),
            pl.BlockSpec((D, FF), lambda: (0, 0)),
            pl.BlockSpec((1, FF), lambda: (0, 0)),
            pl.BlockSpec((FF, D), lambda: (0, 0)),
            pl.BlockSpec((1, D), lambda: (0, 0)),
            pl.BlockSpec((D, OUT), lambda: (0, 0)),
        ],
        out_specs=pl.BlockSpec((B, OUT), lambda: (0, 0)),
        out_shape=jax.ShapeDtypeStruct((B, OUT), jnp.float32),
    )(h, Wm1, bm1.reshape(1, FF), Wm2, bm2.reshape(1, D), Wout)
    return out


# speedup vs baseline: 2.3079x; 1.0060x over previous
import jax
import jax.numpy as jnp
from jax.experimental import pallas as pl
from jax.experimental.pallas import tpu as pltpu

G, S, B, D, FF, OUT = 2, 2048, 128, 1024, 4096, 512
N = S * B
MBLK = 2048


def _extract_kernel(wcat_ref, gidf_ref, xf_ref, yf_ref, tf_ref,
                    out_ref, inpt_ref, hacc_ref):
    mi = pl.program_id(0)
    nm = pl.num_programs(0)

    @pl.when(mi == 0)
    def _prologue():
        m0 = jnp.where(gidf_ref[...] == 0, 1.0, 0.0)
        xr, yr, tr = xf_ref[...], yf_ref[...], tf_ref[...]
        x0 = xr * m0
        y0 = yr * m0
        t0 = tr * m0
        inpt_ref[...] = jnp.concatenate(
            [x0, xr - x0, y0, yr - y0, t0, tr - t0, m0, 1.0 - m0],
            axis=0).astype(jnp.bfloat16)

    HB = MBLK // 2
    w = wcat_ref[...]
    lhs1 = inpt_ref[:, pl.ds(mi * MBLK, HB)]
    lhs2 = inpt_ref[:, pl.ds(mi * MBLK + HB, HB)]
    a1 = jax.lax.dot_general(w, lhs1, (((0,), (0,)), ((), ())),
                             preferred_element_type=jnp.float32)
    a2 = jax.lax.dot_general(w, lhs2, (((0,), (0,)), ((), ())),
                             preferred_element_type=jnp.float32)
    f1 = jnp.tanh(a1[:D, :]) + jnp.maximum(a1[D:, :], 0.0)
    f2 = jnp.tanh(a2[:D, :]) + jnp.maximum(a2[D:, :], 0.0)
    part = f1[:, 0:B] + f2[:, 0:B]
    for j in range(1, HB // B):
        part = part + f1[:, j * B:(j + 1) * B] + f2[:, j * B:(j + 1) * B]

    @pl.when(mi == 0)
    def _init():
        hacc_ref[...] = part

    @pl.when((mi != 0) & (mi != nm - 1))
    def _acc():
        hacc_ref[...] += part

    @pl.when(mi == nm - 1)
    def _emit():
        out_ref[...] = (hacc_ref[...] + part) * (1.0 / S)


def _mlp_kernel(h_ref, wm1_ref, bm1_ref, wm2_ref, bm2_ref, wout_ref, out_ref):
    h1 = jax.lax.dot_general(h_ref[...], wm1_ref[...],
                             (((0,), (0,)), ((), ())),
                             preferred_element_type=jnp.float32)
    h1 = jnp.maximum(h1 + bm1_ref[...], 0.0)
    h2 = jnp.dot(h1, wm2_ref[...],
                 preferred_element_type=jnp.float32) + bm2_ref[...]
    out_ref[...] = jnp.dot(h2, wout_ref[...],
                           preferred_element_type=jnp.float32)


@jax.jit
def kernel(x, y, t, grid_ids, W1, b1, W2, b2, Wm1, bm1, Wm2, bm2, Wout):
    top = jnp.stack([W1[0, 0], W1[1, 0], W1[0, 1], W1[1, 1],
                     W1[0, 2], W1[1, 2], b1[0], b1[1]])
    bot = jnp.stack([W2[0, 0], W2[1, 0], W2[0, 1], W2[1, 1],
                     W2[0, 2], W2[1, 2], b2[0], b2[1]])
    wcat = jnp.concatenate([top, bot], axis=1).astype(jnp.bfloat16)

    xf = x.reshape(1, N)
    yf = y.reshape(1, N)
    tf = t.reshape(1, N)
    gidf = jnp.tile(grid_ids, S).reshape(1, N)

    nm = N // MBLK
    h = pl.pallas_call(
        _extract_kernel,
        grid=(nm,),
        in_specs=[
            pl.BlockSpec((8, 2 * D), lambda mi: (0, 0)),
            pl.BlockSpec((1, N), lambda mi: (0, 0)),
            pl.BlockSpec((1, N), lambda mi: (0, 0)),
            pl.BlockSpec((1, N), lambda mi: (0, 0)),
            pl.BlockSpec((1, N), lambda mi: (0, 0)),
        ],
        out_specs=pl.BlockSpec((D, B), lambda mi: (0, 0)),
        out_shape=jax.ShapeDtypeStruct((D, B), jnp.float32),
        scratch_shapes=[
            pltpu.VMEM((8, N), jnp.bfloat16),
            pltpu.VMEM((D, B), jnp.float32),
        ],
    )(wcat, gidf, xf, yf, tf)

    out = pl.pallas_call(
        _mlp_kernel,
        in_specs=[
            pl.BlockSpec((D, B), lambda: (0, 0)),
            pl.BlockSpec((D, FF), lambda: (0, 0)),
            pl.BlockSpec((1, FF), lambda: (0, 0)),
            pl.BlockSpec((FF, D), lambda: (0, 0)),
            pl.BlockSpec((1, D), lambda: (0, 0)),
            pl.BlockSpec((D, OUT), lambda: (0, 0)),
        ],
        out_specs=pl.BlockSpec((B, OUT), lambda: (0, 0)),
        out_shape=jax.ShapeDtypeStruct((B, OUT), jnp.float32),
    )(h, Wm1, bm1.reshape(1, FF), Wm2, bm2.reshape(1, D), Wout)
    return out
